# Initial kernel scaffold; baseline (speedup 1.0000x reference)
import jax
import jax.numpy as jnp
from jax.experimental import pallas as pl


def kernel(x, Ym):
    raise NotImplementedError("write your pallas kernel here")


# speedup vs baseline: 1.0030x; 1.0030x over previous
import functools

import jax
import jax.numpy as jnp
from jax import lax
from jax.experimental import pallas as pl
from jax.experimental.pallas import tpu as pltpu
from jax.experimental.pallas import tpu_sc as plsc

_B, _C, _H, _W = 4, 96, 224, 224
_D = _H * _W
_KT = 8
_DT = _D // _KT
_NCH = 8
_CHUNK = _D // _NCH
_NROWS = _B * _C * _NCH

_NW = 32
_CPW = _NROWS // _NW
_GL = 8
_NG = _CPW // _GL


def _dist_body(x_ref, y_ref, o_ref, g_acc, x2_acc, y2_acc):
    k = pl.program_id(1)

    @pl.when(k == 0)
    def _():
        g_acc[...] = jnp.zeros_like(g_acc)
        x2_acc[...] = jnp.zeros_like(x2_acc)
        y2_acc[...] = jnp.zeros_like(y2_acc)

    xb = x_ref[0]
    yb = y_ref[0]
    g_acc[...] += lax.dot_general(xb, yb, (((1,), (1,)), ((), ())),
                                  preferred_element_type=jnp.float32)
    x2_acc[...] += jnp.sum(xb * xb, axis=1, keepdims=True)
    y2_acc[...] += jnp.sum(yb * yb, axis=1, keepdims=True)

    @pl.when(k == _KT - 1)
    def _():
        b = pl.program_id(0)
        x2 = x2_acc[...]
        y2 = y2_acc[...].reshape(1, _C)
        d2 = (x2 + y2) - 2.0 * g_acc[...]
        ds = jnp.sqrt(jnp.maximum(d2, 0.0) + 1e-12)
        m = jnp.min(ds, axis=1, keepdims=True)
        ii = lax.broadcasted_iota(jnp.int32, (_C, _C), 1)
        cand = jnp.where(ds <= m, ii, _C)
        idx = jnp.min(cand, axis=1, keepdims=True)
        src = (b * _C + idx) * _NCH
        o_ref[0] = src + lax.broadcasted_iota(jnp.int32, (_C, _NCH), 1)


_dist_call = pl.pallas_call(
    _dist_body,
    grid=(_B, _KT),
    in_specs=[
        pl.BlockSpec((1, _C, _DT), lambda b, k: (b, 0, k)),
        pl.BlockSpec((1, _C, _DT), lambda b, k: (b, 0, k)),
    ],
    out_specs=pl.BlockSpec((1, _C, _NCH), lambda b, k: (b, 0, 0)),
    out_shape=jax.ShapeDtypeStruct((_B, _C, _NCH), jnp.int32),
    scratch_shapes=[
        pltpu.VMEM((_C, _C), jnp.float32),
        pltpu.VMEM((_C, 1), jnp.float32),
        pltpu.VMEM((_C, 1), jnp.float32),
    ],
)


def _gather_body(ym_hbm, idx_hbm, out_hbm, idx_v, buf0, buf1, sem0, sem1):
    wid = lax.axis_index("s") * 2 + lax.axis_index("c")
    base = wid * _CPW
    pltpu.sync_copy(idx_hbm.at[pl.ds(base, _CPW)], idx_v)
    bufs = (buf0, buf1)
    sems = (sem0, sem1)
    pending = [None, None]
    pending[0] = pltpu.async_copy(
        ym_hbm.at[idx_v.at[pl.ds(0, _GL)]], buf0, sem0)
    for g in range(_NG):
        if g + 1 < _NG:
            pending[(g + 1) % 2] = pltpu.async_copy(
                ym_hbm.at[idx_v.at[pl.ds((g + 1) * _GL, _GL)]],
                bufs[(g + 1) % 2], sems[(g + 1) % 2])
        pending[g % 2].wait()
        pltpu.sync_copy(bufs[g % 2], out_hbm.at[pl.ds(base + g * _GL, _GL)])


_gather_call = functools.partial(
    pl.kernel,
    mesh=plsc.VectorSubcoreMesh(core_axis_name="c", subcore_axis_name="s"),
    out_type=jax.ShapeDtypeStruct((_NROWS, _CHUNK), jnp.float32),
    scratch_types=[
        pltpu.VMEM((_CPW,), jnp.int32),
        pltpu.VMEM((_GL, _CHUNK), jnp.float32),
        pltpu.VMEM((_GL, _CHUNK), jnp.float32),
        pltpu.SemaphoreType.DMA,
        pltpu.SemaphoreType.DMA,
    ],
)(_gather_body)


def kernel(x, Ym):
    xf = x.reshape(_B, _C, _D)
    yf = Ym.reshape(_B, _C, _D)
    srcc = _dist_call(xf, yf).reshape(_NROWS)
    ymc = Ym.reshape(_NROWS, _CHUNK)
    out = _gather_call(ymc, srcc)
    return out.reshape(_B, _C, _H, _W)


# speedup vs baseline: 1.1609x; 1.1574x over previous
import functools

import jax
import jax.numpy as jnp
from jax import lax
from jax.experimental import pallas as pl
from jax.experimental.pallas import tpu as pltpu
from jax.experimental.pallas import tpu_sc as plsc

_B, _C, _H, _W = 4, 96, 224, 224
_D = _H * _W
_HT = 56
_KT = _H // _HT
_DT = _HT * _W

_NCH = 8
_CHUNK = _D // _NCH
_NROWS = _B * _C * _NCH

_NW = 32
_CPW = _NROWS // _NW
_GL = 8
_NG = _CPW // _GL


def _dist_body(x_ref, y_ref, o_ref, g_acc, x2_acc, y2_acc):
    k = pl.program_id(1)

    @pl.when(k == 0)
    def _():
        g_acc[...] = jnp.zeros_like(g_acc)
        x2_acc[...] = jnp.zeros_like(x2_acc)
        y2_acc[...] = jnp.zeros_like(y2_acc)

    xb = x_ref[0]
    yb = y_ref[0]
    g = jnp.zeros((_C, _C), jnp.float32)
    for r in range(_HT):
        g += lax.dot_general(xb[:, r, :], yb[:, r * _W:(r + 1) * _W],
                             (((1,), (1,)), ((), ())),
                             preferred_element_type=jnp.float32)
    g_acc[...] += g
    x2_acc[...] += jnp.sum(jnp.sum(xb * xb, axis=2), axis=1, keepdims=True)
    y2_acc[...] += jnp.sum(yb * yb, axis=1, keepdims=True)

    @pl.when(k == _KT - 1)
    def _():
        b = pl.program_id(0)
        x2 = x2_acc[...]
        y2 = y2_acc[...].reshape(1, _C)
        d2 = (x2 + y2) - 2.0 * g_acc[...]
        ds = jnp.sqrt(jnp.maximum(d2, 0.0) + 1e-12)
        m = jnp.min(ds, axis=1, keepdims=True)
        ii = lax.broadcasted_iota(jnp.int32, (_C, _C), 1)
        cand = jnp.where(ds <= m, ii, _C)
        idx = jnp.min(cand, axis=1, keepdims=True)
        src = (b * _C + idx) * _NCH
        o_ref[0] = src + lax.broadcasted_iota(jnp.int32, (_C, _NCH), 1)


_dist_call = pl.pallas_call(
    _dist_body,
    grid=(_B, _KT),
    in_specs=[
        pl.BlockSpec((1, _C, _HT, _W), lambda b, k: (b, 0, k, 0)),
        pl.BlockSpec((1, _C, _DT), lambda b, k: (b, 0, k)),
    ],
    out_specs=pl.BlockSpec((1, _C, _NCH), lambda b, k: (b, 0, 0)),
    out_shape=jax.ShapeDtypeStruct((_B, _C, _NCH), jnp.int32),
    scratch_shapes=[
        pltpu.VMEM((_C, _C), jnp.float32),
        pltpu.VMEM((_C, 1), jnp.float32),
        pltpu.VMEM((_C, 1), jnp.float32),
    ],
)


def _gather_body(ym_hbm, idx_hbm, out_hbm, idx_v, buf0, buf1, sem0, sem1):
    wid = lax.axis_index("s") * 2 + lax.axis_index("c")
    base = wid * _CPW
    pltpu.sync_copy(idx_hbm.at[pl.ds(base, _CPW)], idx_v)
    bufs = (buf0, buf1)
    sems = (sem0, sem1)
    pending = [None, None]
    pending[0] = pltpu.async_copy(
        ym_hbm.at[idx_v.at[pl.ds(0, _GL)]], buf0, sem0)
    for g in range(_NG):
        if g + 1 < _NG:
            pending[(g + 1) % 2] = pltpu.async_copy(
                ym_hbm.at[idx_v.at[pl.ds((g + 1) * _GL, _GL)]],
                bufs[(g + 1) % 2], sems[(g + 1) % 2])
        pending[g % 2].wait()
        pltpu.sync_copy(bufs[g % 2], out_hbm.at[pl.ds(base + g * _GL, _GL)])


_gather_call = functools.partial(
    pl.kernel,
    mesh=plsc.VectorSubcoreMesh(core_axis_name="c", subcore_axis_name="s"),
    out_type=jax.ShapeDtypeStruct((_NROWS, _CHUNK), jnp.float32),
    scratch_types=[
        pltpu.VMEM((_CPW,), jnp.int32),
        pltpu.VMEM((_GL, _CHUNK), jnp.float32),
        pltpu.VMEM((_GL, _CHUNK), jnp.float32),
        pltpu.SemaphoreType.DMA,
        pltpu.SemaphoreType.DMA,
    ],
)(_gather_body)


def kernel(x, Ym):
    yf = Ym.reshape(_B, _C, _D)
    srcc = _dist_call(x, yf).reshape(_NROWS)
    ymc = yf.reshape(_NROWS, _CHUNK)
    out = _gather_call(ymc, srcc)
    return out.reshape(_B, _C, _H, _W)


# speedup vs baseline: 1.2093x; 1.0418x over previous
import functools

import jax
import jax.numpy as jnp
from jax import lax
from jax.experimental import pallas as pl
from jax.experimental.pallas import tpu as pltpu
from jax.experimental.pallas import tpu_sc as plsc

_B, _C, _H, _W = 4, 96, 224, 224
_D = _H * _W
_HT = 56
_KT = _H // _HT
_DT = _HT * _W

_R = _B * _C

_NW = 32
_RPW = _R // _NW


def _dist_body(x_ref, y_ref, o_ref, g_acc, x2_acc, y2_acc):
    k = pl.program_id(1)

    @pl.when(k == 0)
    def _():
        g_acc[...] = jnp.zeros_like(g_acc)
        x2_acc[...] = jnp.zeros_like(x2_acc)
        y2_acc[...] = jnp.zeros_like(y2_acc)

    xb = x_ref[0]
    yb = y_ref[0]
    g = jnp.zeros((_C, _C), jnp.float32)
    for r in range(_HT):
        g += lax.dot_general(xb[:, r, :], yb[:, r * _W:(r + 1) * _W],
                             (((1,), (1,)), ((), ())),
                             preferred_element_type=jnp.float32)
    g_acc[...] += g
    x2_acc[...] += jnp.sum(jnp.sum(xb * xb, axis=2), axis=1, keepdims=True)
    y2_acc[...] += jnp.sum(yb * yb, axis=1, keepdims=True)

    @pl.when(k == _KT - 1)
    def _():
        b = pl.program_id(0)
        x2 = x2_acc[...]
        y2 = y2_acc[...].reshape(1, _C)
        d2 = (x2 + y2) - 2.0 * g_acc[...]
        ds = jnp.sqrt(jnp.maximum(d2, 0.0) + 1e-12)
        m = jnp.min(ds, axis=1, keepdims=True)
        ii = lax.broadcasted_iota(jnp.int32, (_C, _C), 1)
        cand = jnp.where(ds <= m, ii, _C)
        idx = jnp.min(cand, axis=1, keepdims=True)
        o_ref[0] = (b * _C + idx).reshape(1, _C)


_dist_call = pl.pallas_call(
    _dist_body,
    grid=(_B, _KT),
    in_specs=[
        pl.BlockSpec((1, _C, _HT, _W), lambda b, k: (b, 0, k, 0)),
        pl.BlockSpec((1, _C, _DT), lambda b, k: (b, 0, k)),
    ],
    out_specs=pl.BlockSpec((1, 1, _C), lambda b, k: (b, 0, 0)),
    out_shape=jax.ShapeDtypeStruct((_B, 1, _C), jnp.int32),
    scratch_shapes=[
        pltpu.VMEM((_C, _C), jnp.float32),
        pltpu.VMEM((_C, 1), jnp.float32),
        pltpu.VMEM((_C, 1), jnp.float32),
    ],
)


def _gather_body(ym_hbm, idx_hbm, out_hbm, idx_v, buf0, buf1, sem0, sem1):
    wid = lax.axis_index("s") * 2 + lax.axis_index("c")
    base = wid * _RPW
    pltpu.sync_copy(idx_hbm.at[wid], idx_v)
    bufs = (buf0, buf1)
    sems = (sem0, sem1)
    pending = [None, None]
    pending[0] = pltpu.async_copy(ym_hbm.at[idx_v.at[0]], buf0, sem0)
    for g in range(_RPW):
        if g + 1 < _RPW:
            pending[(g + 1) % 2] = pltpu.async_copy(
                ym_hbm.at[idx_v.at[g + 1]], bufs[(g + 1) % 2],
                sems[(g + 1) % 2])
        pending[g % 2].wait()
        pltpu.sync_copy(bufs[g % 2], out_hbm.at[pl.ds(base + g, 1)])


_gather_call = functools.partial(
    pl.kernel,
    mesh=plsc.VectorSubcoreMesh(core_axis_name="c", subcore_axis_name="s"),
    out_type=jax.ShapeDtypeStruct((_R, _D), jnp.float32),
    scratch_types=[
        pltpu.VMEM((_RPW, 1), jnp.int32),
        pltpu.VMEM((1, _D), jnp.float32),
        pltpu.VMEM((1, _D), jnp.float32),
        pltpu.SemaphoreType.DMA,
        pltpu.SemaphoreType.DMA,
    ],
)(_gather_body)


def kernel(x, Ym):
    yf = Ym.reshape(_B, _C, _D)
    srcw = _dist_call(x, yf).reshape(_NW, _RPW, 1)
    ym2 = yf.reshape(_R, _D)
    out = _gather_call(ym2, srcw)
    return out.reshape(_B, _C, _H, _W)


# speedup vs baseline: 1.5269x; 1.2626x over previous
import functools

import jax
import jax.numpy as jnp
from jax import lax
from jax.experimental import pallas as pl
from jax.experimental.pallas import tpu as pltpu
from jax.experimental.pallas import tpu_sc as plsc

_B, _C, _H, _W = 4, 96, 224, 224
_D = _H * _W
_HT = 56
_KT = _H // _HT
_DT = _HT * _W

_R = _B * _C

_NW = 32
_RPW = _R // _NW


def _dist_body(x_ref, y_ref, o_ref, g_acc, x2_acc, y2_acc):
    k = pl.program_id(1)

    @pl.when(k == 0)
    def _():
        g_acc[...] = jnp.zeros_like(g_acc)
        x2_acc[...] = jnp.zeros_like(x2_acc)
        y2_acc[...] = jnp.zeros_like(y2_acc)

    xb = x_ref[0]
    yb = y_ref[...]
    g = jnp.zeros((_C, _C), jnp.float32)
    for r in range(_HT):
        g += lax.dot_general(xb[:, r, :], yb[:, r * _W:(r + 1) * _W],
                             (((1,), (1,)), ((), ())),
                             preferred_element_type=jnp.float32)
    g_acc[...] += g
    x2_acc[...] += jnp.sum(jnp.sum(xb * xb, axis=2), axis=1, keepdims=True)
    y2_acc[...] += jnp.sum(yb * yb, axis=1, keepdims=True)

    @pl.when(k == _KT - 1)
    def _():
        b = pl.program_id(0)
        x2 = x2_acc[...]
        y2 = y2_acc[...].reshape(1, _C)
        d2 = (x2 + y2) - 2.0 * g_acc[...]
        ds = jnp.sqrt(jnp.maximum(d2, 0.0) + 1e-12)
        m = jnp.min(ds, axis=1, keepdims=True)
        ii = lax.broadcasted_iota(jnp.int32, (_C, _C), 1)
        cand = jnp.where(ds <= m, ii, _C)
        idx = jnp.min(cand, axis=1, keepdims=True)
        o_ref[0] = (b * _C + idx).reshape(1, _C)


_dist_call = pl.pallas_call(
    _dist_body,
    grid=(_B, _KT),
    in_specs=[
        pl.BlockSpec((1, _C, _HT, _W), lambda b, k: (b, 0, k, 0)),
        pl.BlockSpec((_C, _DT), lambda b, k: (b, k)),
    ],
    out_specs=pl.BlockSpec((1, 1, _C), lambda b, k: (b, 0, 0)),
    out_shape=jax.ShapeDtypeStruct((_B, 1, _C), jnp.int32),
    scratch_shapes=[
        pltpu.VMEM((_C, _C), jnp.float32),
        pltpu.VMEM((_C, 1), jnp.float32),
        pltpu.VMEM((_C, 1), jnp.float32),
    ],
)


def _gather_body(ym_hbm, idx_hbm, out_hbm, idx_v, buf0, buf1, sem0, sem1):
    wid = lax.axis_index("s") * 2 + lax.axis_index("c")
    base = wid * _RPW
    pltpu.sync_copy(idx_hbm.at[wid], idx_v)
    bufs = (buf0, buf1)
    sems = (sem0, sem1)
    pending = [None, None]
    pending[0] = pltpu.async_copy(ym_hbm.at[idx_v.at[0]], buf0, sem0)
    for g in range(_RPW):
        if g + 1 < _RPW:
            pending[(g + 1) % 2] = pltpu.async_copy(
                ym_hbm.at[idx_v.at[g + 1]], bufs[(g + 1) % 2],
                sems[(g + 1) % 2])
        pending[g % 2].wait()
        pltpu.sync_copy(bufs[g % 2], out_hbm.at[pl.ds(base + g, 1)])


_gather_call = functools.partial(
    pl.kernel,
    mesh=plsc.VectorSubcoreMesh(core_axis_name="c", subcore_axis_name="s"),
    out_type=jax.ShapeDtypeStruct((_R, _D), jnp.float32),
    scratch_types=[
        pltpu.VMEM((_RPW, 1), jnp.int32),
        pltpu.VMEM((1, _D), jnp.float32),
        pltpu.VMEM((1, _D), jnp.float32),
        pltpu.SemaphoreType.DMA,
        pltpu.SemaphoreType.DMA,
    ],
)(_gather_body)


def kernel(x, Ym):
    ym2 = Ym.reshape(_R, _D)
    srcw = _dist_call(x, ym2).reshape(_NW, _RPW, 1)
    out = _gather_call(ym2, srcw)
    return out.reshape(_B, _C, _H, _W)


# speedup vs baseline: 1.6226x; 1.0627x over previous
import functools

import jax
import jax.numpy as jnp
from jax import lax
from jax.experimental import pallas as pl
from jax.experimental.pallas import tpu as pltpu
from jax.experimental.pallas import tpu_sc as plsc

_B, _C, _H, _W = 4, 96, 224, 224
_D = _H * _W
_HT = 56
_KT = _H // _HT
_DT = _HT * _W

_R = _B * _C

_NW = 32
_RPW = _R // _NW


def _dist_body(x_ref, y_ref, o_ref, g_acc, x2_acc, y2_acc):
    k = pl.program_id(1)

    @pl.when(k == 0)
    def _():
        g_acc[...] = jnp.zeros_like(g_acc)
        x2_acc[...] = jnp.zeros_like(x2_acc)
        y2_acc[...] = jnp.zeros_like(y2_acc)

    xb = x_ref[0].reshape(_C, _DT)
    yb = y_ref[...]
    g_acc[...] += lax.dot_general(xb, yb, (((1,), (1,)), ((), ())),
                                  preferred_element_type=jnp.float32)
    x2_acc[...] += jnp.sum(xb * xb, axis=1, keepdims=True)
    y2_acc[...] += jnp.sum(yb * yb, axis=1, keepdims=True)

    @pl.when(k == _KT - 1)
    def _():
        b = pl.program_id(0)
        x2 = x2_acc[...]
        y2 = y2_acc[...].reshape(1, _C)
        d2 = (x2 + y2) - 2.0 * g_acc[...]
        ds = jnp.sqrt(jnp.maximum(d2, 0.0) + 1e-12)
        m = jnp.min(ds, axis=1, keepdims=True)
        ii = lax.broadcasted_iota(jnp.int32, (_C, _C), 1)
        cand = jnp.where(ds <= m, ii, _C)
        idx = jnp.min(cand, axis=1, keepdims=True)
        o_ref[0] = (b * _C + idx).reshape(1, _C)


_dist_call = pl.pallas_call(
    _dist_body,
    grid=(_B, _KT),
    in_specs=[
        pl.BlockSpec((1, _C, _HT, _W), lambda b, k: (b, 0, k, 0)),
        pl.BlockSpec((_C, _DT), lambda b, k: (b, k)),
    ],
    out_specs=pl.BlockSpec((1, 1, _C), lambda b, k: (b, 0, 0)),
    out_shape=jax.ShapeDtypeStruct((_B, 1, _C), jnp.int32),
    scratch_shapes=[
        pltpu.VMEM((_C, _C), jnp.float32),
        pltpu.VMEM((_C, 1), jnp.float32),
        pltpu.VMEM((_C, 1), jnp.float32),
    ],
)


def _gather_body(ym_hbm, idx_hbm, out_hbm, idx_v, buf0, buf1, sem0, sem1):
    wid = lax.axis_index("s") * 2 + lax.axis_index("c")
    base = wid * _RPW
    pltpu.sync_copy(idx_hbm.at[wid], idx_v)
    bufs = (buf0, buf1)
    sems = (sem0, sem1)
    pending = [None, None]
    pending[0] = pltpu.async_copy(ym_hbm.at[idx_v.at[0]], buf0, sem0)
    for g in range(_RPW):
        if g + 1 < _RPW:
            pending[(g + 1) % 2] = pltpu.async_copy(
                ym_hbm.at[idx_v.at[g + 1]], bufs[(g + 1) % 2],
                sems[(g + 1) % 2])
        pending[g % 2].wait()
        pltpu.sync_copy(bufs[g % 2], out_hbm.at[pl.ds(base + g, 1)])


_gather_call = functools.partial(
    pl.kernel,
    mesh=plsc.VectorSubcoreMesh(core_axis_name="c", subcore_axis_name="s"),
    out_type=jax.ShapeDtypeStruct((_R, _D), jnp.float32),
    scratch_types=[
        pltpu.VMEM((_RPW, 1), jnp.int32),
        pltpu.VMEM((1, _D), jnp.float32),
        pltpu.VMEM((1, _D), jnp.float32),
        pltpu.SemaphoreType.DMA,
        pltpu.SemaphoreType.DMA,
    ],
)(_gather_body)


def kernel(x, Ym):
    ym2 = Ym.reshape(_R, _D)
    srcw = _dist_call(x, ym2).reshape(_NW, _RPW, 1)
    out = _gather_call(ym2, srcw)
    return out.reshape(_B, _C, _H, _W)


# speedup vs baseline: 1.9671x; 1.2123x over previous
import functools

import jax
import jax.numpy as jnp
from jax import lax
from jax.experimental import pallas as pl
from jax.experimental.pallas import tpu as pltpu
from jax.experimental.pallas import tpu_sc as plsc

_B, _C, _H, _W = 4, 96, 224, 224
_D = _H * _W
_HT = 56
_KT = _H // _HT
_DT = _HT * _W

_R = _B * _C

_NW = 32
_RPW = _R // _NW


def _dist_body(x_ref, y_ref, o_ref, oy_ref, g_acc, x2_acc, y2_acc):
    k = pl.program_id(1)

    @pl.when(k == 0)
    def _():
        g_acc[...] = jnp.zeros_like(g_acc)
        x2_acc[...] = jnp.zeros_like(x2_acc)
        y2_acc[...] = jnp.zeros_like(y2_acc)

    xb = x_ref[0].reshape(_C, _DT)
    yb = y_ref[0].reshape(_C, _DT)
    oy_ref[...] = yb
    g_acc[...] += lax.dot_general(xb, yb, (((1,), (1,)), ((), ())),
                                  preferred_element_type=jnp.float32)
    x2_acc[...] += jnp.sum(xb * xb, axis=1, keepdims=True)
    y2_acc[...] += jnp.sum(yb * yb, axis=1, keepdims=True)

    @pl.when(k == _KT - 1)
    def _():
        b = pl.program_id(0)
        x2 = x2_acc[...]
        y2 = y2_acc[...].reshape(1, _C)
        d2 = (x2 + y2) - 2.0 * g_acc[...]
        ds = jnp.sqrt(jnp.maximum(d2, 0.0) + 1e-12)
        m = jnp.min(ds, axis=1, keepdims=True)
        ii = lax.broadcasted_iota(jnp.int32, (_C, _C), 1)
        cand = jnp.where(ds <= m, ii, _C)
        idx = jnp.min(cand, axis=1, keepdims=True)
        o_ref[0] = (b * _C + idx).reshape(1, _C)


_dist_call = pl.pallas_call(
    _dist_body,
    grid=(_B, _KT),
    in_specs=[
        pl.BlockSpec((1, _C, _HT, _W), lambda b, k: (b, 0, k, 0)),
        pl.BlockSpec((1, _C, _HT, _W), lambda b, k: (b, 0, k, 0)),
    ],
    out_specs=[
        pl.BlockSpec((1, 1, _C), lambda b, k: (b, 0, 0)),
        pl.BlockSpec((_C, _DT), lambda b, k: (b, k)),
    ],
    out_shape=[
        jax.ShapeDtypeStruct((_B, 1, _C), jnp.int32),
        jax.ShapeDtypeStruct((_R, _D), jnp.float32),
    ],
    scratch_shapes=[
        pltpu.VMEM((_C, _C), jnp.float32),
        pltpu.VMEM((_C, 1), jnp.float32),
        pltpu.VMEM((_C, 1), jnp.float32),
    ],
)


def _gather_body(ym_hbm, idx_hbm, out_hbm, idx_v, buf0, buf1, sem0, sem1):
    wid = lax.axis_index("s") * 2 + lax.axis_index("c")
    base = wid * _RPW
    pltpu.sync_copy(idx_hbm.at[wid], idx_v)
    bufs = (buf0, buf1)
    sems = (sem0, sem1)
    pending = [None, None]
    pending[0] = pltpu.async_copy(ym_hbm.at[idx_v.at[0]], buf0, sem0)
    for g in range(_RPW):
        if g + 1 < _RPW:
            pending[(g + 1) % 2] = pltpu.async_copy(
                ym_hbm.at[idx_v.at[g + 1]], bufs[(g + 1) % 2],
                sems[(g + 1) % 2])
        pending[g % 2].wait()
        pltpu.sync_copy(bufs[g % 2], out_hbm.at[pl.ds(base + g, 1)])


_gather_call = functools.partial(
    pl.kernel,
    mesh=plsc.VectorSubcoreMesh(core_axis_name="c", subcore_axis_name="s"),
    out_type=jax.ShapeDtypeStruct((_R, _D), jnp.float32),
    scratch_types=[
        pltpu.VMEM((_RPW, 1), jnp.int32),
        pltpu.VMEM((1, _D), jnp.float32),
        pltpu.VMEM((1, _D), jnp.float32),
        pltpu.SemaphoreType.DMA,
        pltpu.SemaphoreType.DMA,
    ],
)(_gather_body)


def kernel(x, Ym):
    src, ym2 = _dist_call(x, Ym)
    out = _gather_call(ym2, src.reshape(_NW, _RPW, 1))
    return out.reshape(_B, _C, _H, _W)
